# baseline (speedup 1.0000x reference)
import jax
import jax.numpy as jnp
from jax.experimental import pallas as pl
from jax.experimental.pallas import tpu as pltpu

TOP_K = 8
NUM_EXPERTS = 64
HIDDEN = 768
M_BLK = 2048
NEG_INF = float("-inf")


def _router_block(h_ref, wT_ref, logits_ref, rw_ref, se_ref):
    h = h_ref[...]
    wT = wT_ref[...]
    logits = jnp.dot(h, wT, preferred_element_type=jnp.float32)
    logits_ref[...] = logits

    work = logits.T
    row_f = jax.lax.broadcasted_iota(
        jnp.int32, (NUM_EXPERTS, M_BLK), 0).astype(jnp.float32)
    vals, idxs = [], []
    for _ in range(TOP_K):
        m = jnp.max(work, axis=0, keepdims=True)
        hit = work == m
        idx_f = jnp.min(jnp.where(hit, row_f, jnp.float32(NUM_EXPERTS)),
                        axis=0, keepdims=True)
        vals.append(m)
        idxs.append(idx_f)
        work = jnp.where(row_f == idx_f, NEG_INF, work)

    v = jnp.concatenate(vals, axis=0)
    i_f = jnp.concatenate(idxs, axis=0)
    e = jnp.exp(v - v[0:1, :])
    rw = e / jnp.sum(e, axis=0, keepdims=True)
    rw_ref[...] = rw.T
    se_ref[...] = i_f.T.astype(jnp.int32)


def kernel(hidden_states, gate_weight):
    m = hidden_states.shape[0]
    grid = (m // M_BLK,)
    wT = gate_weight.T
    out_shapes = (
        jax.ShapeDtypeStruct((m, NUM_EXPERTS), jnp.float32),
        jax.ShapeDtypeStruct((m, TOP_K), jnp.float32),
        jax.ShapeDtypeStruct((m, TOP_K), jnp.int32),
    )
    return pl.pallas_call(
        _router_block,
        grid=grid,
        in_specs=[
            pl.BlockSpec((M_BLK, HIDDEN), lambda i: (i, 0)),
            pl.BlockSpec((HIDDEN, NUM_EXPERTS), lambda i: (0, 0)),
        ],
        out_specs=(
            pl.BlockSpec((M_BLK, NUM_EXPERTS), lambda i: (i, 0)),
            pl.BlockSpec((M_BLK, TOP_K), lambda i: (i, 0)),
            pl.BlockSpec((M_BLK, TOP_K), lambda i: (i, 0)),
        ),
        out_shape=out_shapes,
        compiler_params=pltpu.CompilerParams(
            dimension_semantics=("parallel",),
        ),
    )(hidden_states, wT)


# speedup vs baseline: 1.0395x; 1.0395x over previous
import jax
import jax.numpy as jnp
from jax.experimental import pallas as pl
from jax.experimental.pallas import tpu as pltpu

TOP_K = 8
NUM_EXPERTS = 64
HIDDEN = 768
M_BLK = 4096
NEG_INF = float("-inf")


def _router_block(h_ref, wT_ref, logits_ref, rw_ref, se_ref):
    h = h_ref[...]
    wT = wT_ref[...]
    logits = jnp.dot(h, wT, preferred_element_type=jnp.float32)
    logits_ref[...] = logits

    work = logits.T
    row_f = jax.lax.broadcasted_iota(
        jnp.int32, (NUM_EXPERTS, M_BLK), 0).astype(jnp.float32)
    vals, idxs = [], []
    for _ in range(TOP_K):
        m = jnp.max(work, axis=0, keepdims=True)
        hit = work == m
        idx_f = jnp.min(jnp.where(hit, row_f, jnp.float32(NUM_EXPERTS)),
                        axis=0, keepdims=True)
        vals.append(m)
        idxs.append(idx_f)
        work = jnp.where(row_f == idx_f, NEG_INF, work)

    v = jnp.concatenate(vals, axis=0)
    i_f = jnp.concatenate(idxs, axis=0)
    e = jnp.exp(v - v[0:1, :])
    rw = e / jnp.sum(e, axis=0, keepdims=True)
    rw_ref[...] = rw.T
    se_ref[...] = i_f.T.astype(jnp.int32)


def kernel(hidden_states, gate_weight):
    m = hidden_states.shape[0]
    grid = (m // M_BLK,)
    wT = gate_weight.T
    out_shapes = (
        jax.ShapeDtypeStruct((m, NUM_EXPERTS), jnp.float32),
        jax.ShapeDtypeStruct((m, TOP_K), jnp.float32),
        jax.ShapeDtypeStruct((m, TOP_K), jnp.int32),
    )
    return pl.pallas_call(
        _router_block,
        grid=grid,
        in_specs=[
            pl.BlockSpec((M_BLK, HIDDEN), lambda i: (i, 0)),
            pl.BlockSpec((HIDDEN, NUM_EXPERTS), lambda i: (0, 0)),
        ],
        out_specs=(
            pl.BlockSpec((M_BLK, NUM_EXPERTS), lambda i: (i, 0)),
            pl.BlockSpec((M_BLK, TOP_K), lambda i: (i, 0)),
            pl.BlockSpec((M_BLK, TOP_K), lambda i: (i, 0)),
        ),
        out_shape=out_shapes,
        compiler_params=pltpu.CompilerParams(
            dimension_semantics=("parallel",),
        ),
    )(hidden_states, wT)


# speedup vs baseline: 1.0411x; 1.0015x over previous
import jax
import jax.numpy as jnp
from jax.experimental import pallas as pl
from jax.experimental.pallas import tpu as pltpu

TOP_K = 8
NUM_EXPERTS = 64
HIDDEN = 768
M_BLK = 4096
NEG_INF = float("-inf")


def _router_block(h_ref, wT_ref, logits_ref, rw_ref, se_ref):
    h = h_ref[...]
    wT = wT_ref[...]
    logits = jnp.dot(h, wT, preferred_element_type=jnp.float32)
    logits_ref[...] = logits

    work = logits.T
    row_f = jax.lax.broadcasted_iota(
        jnp.int32, (NUM_EXPERTS, M_BLK), 0).astype(jnp.float32)
    vals, idxs = [], []
    for _ in range(TOP_K):
        m = jnp.max(work, axis=0, keepdims=True)
        hit = work == m
        idx_f = jnp.min(jnp.where(hit, row_f, jnp.float32(NUM_EXPERTS)),
                        axis=0, keepdims=True)
        vals.append(m)
        idxs.append(idx_f)
        work = jnp.where(row_f == idx_f, NEG_INF, work)

    v = jnp.concatenate(vals, axis=0)
    i_f = jnp.concatenate(idxs, axis=0)
    e = jnp.exp(v - v[0:1, :])
    rw = e / jnp.sum(e, axis=0, keepdims=True)
    rw_ref[...] = rw.T
    se_ref[...] = i_f.T.astype(jnp.int32)


def kernel(hidden_states, gate_weight):
    m = hidden_states.shape[0]
    grid = (m // M_BLK,)
    wT = gate_weight.T
    out_shapes = (
        jax.ShapeDtypeStruct((m, NUM_EXPERTS), jnp.float32),
        jax.ShapeDtypeStruct((m, TOP_K), jnp.float32),
        jax.ShapeDtypeStruct((m, TOP_K), jnp.int32),
    )
    return pl.pallas_call(
        _router_block,
        grid=grid,
        in_specs=[
            pl.BlockSpec((M_BLK, HIDDEN), lambda i: (i, 0)),
            pl.BlockSpec((HIDDEN, NUM_EXPERTS), lambda i: (0, 0)),
        ],
        out_specs=(
            pl.BlockSpec((M_BLK, NUM_EXPERTS), lambda i: (i, 0)),
            pl.BlockSpec((M_BLK, TOP_K), lambda i: (i, 0)),
            pl.BlockSpec((M_BLK, TOP_K), lambda i: (i, 0)),
        ),
        out_shape=out_shapes,
        compiler_params=pltpu.CompilerParams(
            dimension_semantics=("arbitrary",),
        ),
    )(hidden_states, wT)
